# Initial kernel scaffold; baseline (speedup 1.0000x reference)
import functools

import jax
import jax.numpy as jnp
from jax import lax
from jax.experimental import pallas as pl
from jax.experimental.pallas import tpu as pltpu
from jax.experimental.pallas import tpu_sc as plsc

VOCAB = 100000
EMB = 128
HID = 256
BATCH = 1024
SEQ = 50

N_TOK = BATCH * SEQ
NW = 32
N_CHUNK = 4
CHUNK = N_TOK // (NW * N_CHUNK)


def _make_sc_gather():
    mesh = plsc.VectorSubcoreMesh(core_axis_name="c", subcore_axis_name="s")

    @functools.partial(
        pl.kernel,
        out_type=jax.ShapeDtypeStruct((N_TOK, EMB), jnp.float32),
        mesh=mesh,
        scratch_types=[
            pltpu.VMEM((N_CHUNK, CHUNK), jnp.int32),
            pltpu.VMEM((CHUNK, EMB), jnp.float32),
            pltpu.SemaphoreType.DMA,
        ],
    )
    def gather_kernel(table_hbm, idx_hbm, out_hbm, idx_v, rows_v, sem):
        wid = lax.axis_index("s") * 2 + lax.axis_index("c")
        base = wid * (N_CHUNK * CHUNK)
        pltpu.sync_copy(idx_hbm.at[wid], idx_v)

        def chunk_body(j, carry):
            pltpu.async_copy(table_hbm.at[idx_v.at[j]], rows_v, sem).wait()
            pltpu.sync_copy(rows_v, out_hbm.at[pl.ds(base + j * CHUNK, CHUNK)])
            return carry

        lax.fori_loop(0, N_CHUNK, chunk_body, 0, unroll=False)

    return gather_kernel


_sc_gather = _make_sc_gather()


def _lstm_scan_kernel(emb_ref, wih0_ref, whh0_ref, b0_ref,
                      wih1_ref, whh1_ref, b1_ref,
                      hidden_ref, cell_ref,
                      h0_ref, c0_ref, h1_ref, c1_ref):
    t = pl.program_id(0)

    @pl.when(t == 0)
    def _init():
        h0_ref[...] = jnp.zeros_like(h0_ref)
        c0_ref[...] = jnp.zeros_like(c0_ref)
        h1_ref[...] = jnp.zeros_like(h1_ref)
        c1_ref[...] = jnp.zeros_like(c1_ref)

    x = emb_ref[0]

    def cell(x_t, h, c, wih, whh, b):
        gates = (jnp.dot(x_t, wih, preferred_element_type=jnp.float32)
                 + jnp.dot(h, whh, preferred_element_type=jnp.float32)
                 + b)
        i = jax.nn.sigmoid(gates[:, 0 * HID:1 * HID])
        f = jax.nn.sigmoid(gates[:, 1 * HID:2 * HID])
        g = jnp.tanh(gates[:, 2 * HID:3 * HID])
        o = jax.nn.sigmoid(gates[:, 3 * HID:4 * HID])
        c_new = f * c + i * g
        h_new = o * jnp.tanh(c_new)
        return h_new, c_new

    h0, c0 = cell(x, h0_ref[...], c0_ref[...], wih0_ref[...], whh0_ref[...],
                  b0_ref[0])
    h0_ref[...] = h0
    c0_ref[...] = c0
    h1, c1 = cell(h0, h1_ref[...], c1_ref[...], wih1_ref[...], whh1_ref[...],
                  b1_ref[0])
    h1_ref[...] = h1
    c1_ref[...] = c1

    @pl.when(t == SEQ - 1)
    def _emit():
        hidden_ref[0] = h0
        hidden_ref[1] = h1
        cell_ref[0] = c0
        cell_ref[1] = c1


def _lstm_scan(emb_tm, wih0_t, whh0_t, b0, wih1_t, whh1_t, b1,
               interpret=False):
    full = lambda shape: pl.BlockSpec(shape, lambda t: (0,) * len(shape))
    return pl.pallas_call(
        _lstm_scan_kernel,
        grid=(SEQ,),
        in_specs=[
            pl.BlockSpec((1, BATCH, EMB), lambda t: (t, 0, 0)),
            full((EMB, 4 * HID)),
            full((HID, 4 * HID)),
            full((1, 4 * HID)),
            full((HID, 4 * HID)),
            full((HID, 4 * HID)),
            full((1, 4 * HID)),
        ],
        out_specs=[
            full((2, BATCH, HID)),
            full((2, BATCH, HID)),
        ],
        out_shape=[
            jax.ShapeDtypeStruct((2, BATCH, HID), jnp.float32),
            jax.ShapeDtypeStruct((2, BATCH, HID), jnp.float32),
        ],
        scratch_shapes=[pltpu.VMEM((BATCH, HID), jnp.float32)] * 4,
        interpret=interpret,
    )(emb_tm, wih0_t, whh0_t, b0, wih1_t, whh1_t, b1)


@jax.jit
def kernel(table, W_ih_0, W_hh_0, b_ih_0, b_hh_0,
           W_ih_1, W_hh_1, b_ih_1, b_hh_1, x):
    idx = x.astype(jnp.int32).T.reshape(NW, N_CHUNK, CHUNK)
    emb = _sc_gather(table, idx)
    emb_tm = emb.reshape(SEQ, BATCH, EMB)

    hidden, cell = _lstm_scan(
        emb_tm,
        W_ih_0.T, W_hh_0.T, (b_ih_0 + b_hh_0).reshape(1, -1),
        W_ih_1.T, W_hh_1.T, (b_ih_1 + b_hh_1).reshape(1, -1),
    )
    return (hidden, cell)


# speedup vs baseline: 3.8281x; 3.8281x over previous
import functools

import jax
import jax.numpy as jnp
from jax import lax
from jax.experimental import pallas as pl
from jax.experimental.pallas import tpu as pltpu
from jax.experimental.pallas import tpu_sc as plsc

VOCAB = 100000
EMB = 128
HID = 256
BATCH = 1024
SEQ = 50

N_TOK = BATCH * SEQ
NW = 32
N_CHUNK = 4
CHUNK = N_TOK // (NW * N_CHUNK)


@functools.cache
def _make_sc_gather():
    mesh = plsc.VectorSubcoreMesh(core_axis_name="c", subcore_axis_name="s")

    @functools.partial(
        pl.kernel,
        out_type=jax.ShapeDtypeStruct((N_TOK, EMB), jnp.float32),
        mesh=mesh,
        scratch_types=[
            pltpu.VMEM((CHUNK,), jnp.int32),
            pltpu.VMEM((CHUNK, EMB), jnp.float32),
            pltpu.SemaphoreType.DMA,
        ],
    )
    def gather_kernel(table_hbm, idx_hbm, out_hbm, idx_c, rows_v, sem):
        wid = lax.axis_index("s") * 2 + lax.axis_index("c")
        base = wid * (N_CHUNK * CHUNK)

        def chunk_body(j, carry):
            pltpu.sync_copy(idx_hbm.at[wid * N_CHUNK + j], idx_c)
            pltpu.async_copy(table_hbm.at[idx_c], rows_v, sem).wait()
            pltpu.sync_copy(rows_v, out_hbm.at[pl.ds(base + j * CHUNK, CHUNK)])
            return carry

        lax.fori_loop(0, N_CHUNK, chunk_body, 0, unroll=False)

    return gather_kernel


def _lstm_scan_kernel(emb_ref, wih0_ref, whh0_ref, b0_ref,
                      wih1_ref, whh1_ref, b1_ref,
                      hidden_ref, cell_ref,
                      h0_ref, c0_ref, h1_ref, c1_ref):
    t = pl.program_id(0)

    @pl.when(t == 0)
    def _init():
        h0_ref[...] = jnp.zeros_like(h0_ref)
        c0_ref[...] = jnp.zeros_like(c0_ref)
        h1_ref[...] = jnp.zeros_like(h1_ref)
        c1_ref[...] = jnp.zeros_like(c1_ref)

    x = emb_ref[0]

    def cell(x_t, h, c, wih, whh, b):
        gates = (jnp.dot(x_t, wih, preferred_element_type=jnp.float32)
                 + jnp.dot(h, whh, preferred_element_type=jnp.float32)
                 + b)
        i = jax.nn.sigmoid(gates[:, 0 * HID:1 * HID])
        f = jax.nn.sigmoid(gates[:, 1 * HID:2 * HID])
        g = jnp.tanh(gates[:, 2 * HID:3 * HID])
        o = jax.nn.sigmoid(gates[:, 3 * HID:4 * HID])
        c_new = f * c + i * g
        h_new = o * jnp.tanh(c_new)
        return h_new, c_new

    h0, c0 = cell(x, h0_ref[...], c0_ref[...], wih0_ref[...], whh0_ref[...],
                  b0_ref[0])
    h0_ref[...] = h0
    c0_ref[...] = c0
    h1, c1 = cell(h0, h1_ref[...], c1_ref[...], wih1_ref[...], whh1_ref[...],
                  b1_ref[0])
    h1_ref[...] = h1
    c1_ref[...] = c1

    @pl.when(t == SEQ - 1)
    def _emit():
        hidden_ref[0] = h0
        hidden_ref[1] = h1
        cell_ref[0] = c0
        cell_ref[1] = c1


def _lstm_scan(emb_tm, wih0_t, whh0_t, b0, wih1_t, whh1_t, b1,
               interpret=False):
    full = lambda shape: pl.BlockSpec(shape, lambda t: (0,) * len(shape))
    return pl.pallas_call(
        _lstm_scan_kernel,
        grid=(SEQ,),
        in_specs=[
            pl.BlockSpec((1, BATCH, EMB), lambda t: (t, 0, 0)),
            full((EMB, 4 * HID)),
            full((HID, 4 * HID)),
            full((1, 4 * HID)),
            full((HID, 4 * HID)),
            full((HID, 4 * HID)),
            full((1, 4 * HID)),
        ],
        out_specs=[
            full((2, BATCH, HID)),
            full((2, BATCH, HID)),
        ],
        out_shape=[
            jax.ShapeDtypeStruct((2, BATCH, HID), jnp.float32),
            jax.ShapeDtypeStruct((2, BATCH, HID), jnp.float32),
        ],
        scratch_shapes=[pltpu.VMEM((BATCH, HID), jnp.float32)] * 4,
        interpret=interpret,
    )(emb_tm, wih0_t, whh0_t, b0, wih1_t, whh1_t, b1)


@jax.jit
def kernel(table, W_ih_0, W_hh_0, b_ih_0, b_hh_0,
           W_ih_1, W_hh_1, b_ih_1, b_hh_1, x):
    idx = x.astype(jnp.int32).T.reshape(NW * N_CHUNK, CHUNK)
    emb = _make_sc_gather()(table, idx)
    emb_tm = emb.reshape(SEQ, BATCH, EMB)

    hidden, cell = _lstm_scan(
        emb_tm,
        W_ih_0.T, W_hh_0.T, (b_ih_0 + b_hh_0).reshape(1, -1),
        W_ih_1.T, W_hh_1.T, (b_ih_1 + b_hh_1).reshape(1, -1),
    )
    return (hidden, cell)


# speedup vs baseline: 4.3623x; 1.1395x over previous
import functools

import jax
import jax.numpy as jnp
from jax import lax
from jax.experimental import pallas as pl
from jax.experimental.pallas import tpu as pltpu
from jax.experimental.pallas import tpu_sc as plsc

VOCAB = 100000
EMB = 128
HID = 256
BATCH = 1024
SEQ = 50

N_TOK = BATCH * SEQ
NW = 32
N_CHUNK = 4
CHUNK = N_TOK // (NW * N_CHUNK)


@functools.cache
def _make_sc_gather():
    mesh = plsc.VectorSubcoreMesh(core_axis_name="c", subcore_axis_name="s")

    @functools.partial(
        pl.kernel,
        out_type=jax.ShapeDtypeStruct((N_TOK, EMB), jnp.float32),
        mesh=mesh,
        scratch_types=[
            pltpu.VMEM((CHUNK,), jnp.int32),
            pltpu.VMEM((CHUNK, EMB), jnp.float32),
            pltpu.SemaphoreType.DMA,
        ],
    )
    def gather_kernel(table_hbm, idx_hbm, out_hbm, idx_c, rows_v, sem):
        wid = lax.axis_index("s") * 2 + lax.axis_index("c")
        base = wid * (N_CHUNK * CHUNK)

        def chunk_body(j, carry):
            pltpu.sync_copy(idx_hbm.at[wid * N_CHUNK + j], idx_c)
            pltpu.async_copy(table_hbm.at[idx_c], rows_v, sem).wait()
            pltpu.sync_copy(rows_v, out_hbm.at[pl.ds(base + j * CHUNK, CHUNK)])
            return carry

        lax.fori_loop(0, N_CHUNK, chunk_body, 0, unroll=False)

    return gather_kernel


def _lstm_scan_kernel(emb_ref, w0_ref, b0_ref, w1_ref, b1_ref,
                      hidden_ref, cell_ref,
                      h0_ref, c0_ref, h1_ref, c1_ref):
    t = pl.program_id(0)
    bf = jnp.bfloat16

    @pl.when(t == 0)
    def _init():
        h0_ref[...] = jnp.zeros_like(h0_ref)
        c0_ref[...] = jnp.zeros_like(c0_ref)
        h1_ref[...] = jnp.zeros_like(h1_ref)
        c1_ref[...] = jnp.zeros_like(c1_ref)

    def sig(v):
        return 0.5 * jnp.tanh(0.5 * v) + 0.5

    def cell(xh, c, w, b):
        gates = jnp.dot(xh, w, preferred_element_type=jnp.float32) + b
        i = sig(gates[:, 0 * HID:1 * HID])
        f = sig(gates[:, 1 * HID:2 * HID])
        g = jnp.tanh(gates[:, 2 * HID:3 * HID])
        o = sig(gates[:, 3 * HID:4 * HID])
        c_new = f * c + i * g
        h_new = o * jnp.tanh(c_new)
        return h_new, c_new

    x = emb_ref[0].astype(bf)
    xh0 = jnp.concatenate([x, h0_ref[...]], axis=1)
    h0, c0 = cell(xh0, c0_ref[...], w0_ref[...], b0_ref[0])
    h0_ref[...] = h0.astype(bf)
    c0_ref[...] = c0
    xh1 = jnp.concatenate([h0.astype(bf), h1_ref[...]], axis=1)
    h1, c1 = cell(xh1, c1_ref[...], w1_ref[...], b1_ref[0])
    h1_ref[...] = h1.astype(bf)
    c1_ref[...] = c1

    @pl.when(t == SEQ - 1)
    def _emit():
        hidden_ref[0] = h0
        hidden_ref[1] = h1
        cell_ref[0] = c0
        cell_ref[1] = c1


def _lstm_scan(emb_tm, w0, b0, w1, b1, interpret=False):
    full = lambda shape: pl.BlockSpec(shape, lambda t: (0,) * len(shape))
    return pl.pallas_call(
        _lstm_scan_kernel,
        grid=(SEQ,),
        in_specs=[
            pl.BlockSpec((1, BATCH, EMB), lambda t: (t, 0, 0)),
            full((EMB + HID, 4 * HID)),
            full((1, 4 * HID)),
            full((2 * HID, 4 * HID)),
            full((1, 4 * HID)),
        ],
        out_specs=[
            full((2, BATCH, HID)),
            full((2, BATCH, HID)),
        ],
        out_shape=[
            jax.ShapeDtypeStruct((2, BATCH, HID), jnp.float32),
            jax.ShapeDtypeStruct((2, BATCH, HID), jnp.float32),
        ],
        scratch_shapes=[
            pltpu.VMEM((BATCH, HID), jnp.bfloat16),
            pltpu.VMEM((BATCH, HID), jnp.float32),
            pltpu.VMEM((BATCH, HID), jnp.bfloat16),
            pltpu.VMEM((BATCH, HID), jnp.float32),
        ],
        interpret=interpret,
    )(emb_tm, w0, b0, w1, b1)


@jax.jit
def kernel(table, W_ih_0, W_hh_0, b_ih_0, b_hh_0,
           W_ih_1, W_hh_1, b_ih_1, b_hh_1, x):
    idx = x.astype(jnp.int32).T.reshape(NW * N_CHUNK, CHUNK)
    emb = _make_sc_gather()(table, idx)
    emb_tm = emb.reshape(SEQ, BATCH, EMB)

    bf = jnp.bfloat16
    w0 = jnp.concatenate([W_ih_0.T, W_hh_0.T], axis=0).astype(bf)
    w1 = jnp.concatenate([W_ih_1.T, W_hh_1.T], axis=0).astype(bf)
    hidden, cell = _lstm_scan(
        emb_tm,
        w0, (b_ih_0 + b_hh_0).reshape(1, -1),
        w1, (b_ih_1 + b_hh_1).reshape(1, -1),
    )
    return (hidden, cell)


# speedup vs baseline: 4.7944x; 1.0991x over previous
import functools

import jax
import jax.numpy as jnp
from jax import lax
from jax.experimental import pallas as pl
from jax.experimental.pallas import tpu as pltpu
from jax.experimental.pallas import tpu_sc as plsc

VOCAB = 100000
EMB = 128
HID = 256
BATCH = 1024
SEQ = 50

N_TOK = BATCH * SEQ
NW = 32
N_CHUNK = 4
CHUNK = N_TOK // (NW * N_CHUNK)


@functools.cache
def _make_sc_gather():
    mesh = plsc.VectorSubcoreMesh(core_axis_name="c", subcore_axis_name="s")

    @functools.partial(
        pl.kernel,
        out_type=jax.ShapeDtypeStruct((N_TOK, EMB), jnp.float32),
        mesh=mesh,
        scratch_types=[
            pltpu.VMEM((CHUNK,), jnp.int32),
            pltpu.VMEM((CHUNK, EMB), jnp.float32),
            pltpu.SemaphoreType.DMA,
        ],
    )
    def gather_kernel(table_hbm, idx_hbm, out_hbm, idx_c, rows_v, sem):
        wid = lax.axis_index("s") * 2 + lax.axis_index("c")
        base = wid * (N_CHUNK * CHUNK)

        def chunk_body(j, carry):
            pltpu.sync_copy(idx_hbm.at[wid * N_CHUNK + j], idx_c)
            pltpu.async_copy(table_hbm.at[idx_c], rows_v, sem).wait()
            pltpu.sync_copy(rows_v, out_hbm.at[pl.ds(base + j * CHUNK, CHUNK)])
            return carry

        lax.fori_loop(0, N_CHUNK, chunk_body, 0, unroll=False)

    return gather_kernel


K0 = EMB + HID
K1 = 2 * HID
NB = 2
BB = BATCH // NB


def _lstm_scan_kernel(emb_ref, w0_ref, b0_ref, w1_ref, b1_ref,
                      hidden_ref, cell_ref,
                      xh0_ref, c0_ref, xh1_ref, c1_ref):
    t = pl.program_id(0)
    bf = jnp.bfloat16

    @pl.when(t == 0)
    def _init():
        xh0_ref[:, EMB:] = jnp.zeros((BATCH, HID), bf)
        xh1_ref[...] = jnp.zeros((BATCH, K1), bf)
        c0_ref[...] = jnp.zeros_like(c0_ref)
        c1_ref[...] = jnp.zeros_like(c1_ref)

    def cell(xh, c, w, b):
        gates = jnp.dot(xh, w, preferred_element_type=jnp.float32)
        i = 0.5 * jnp.tanh(gates[:, 0 * HID:1 * HID] + b[0 * HID:1 * HID]) + 0.5
        f = 0.5 * jnp.tanh(gates[:, 1 * HID:2 * HID] + b[1 * HID:2 * HID]) + 0.5
        g = jnp.tanh(gates[:, 2 * HID:3 * HID] + b[2 * HID:3 * HID])
        o = 0.5 * jnp.tanh(gates[:, 3 * HID:4 * HID] + b[3 * HID:4 * HID]) + 0.5
        c_new = f * c + i * g
        h_new = o * jnp.tanh(c_new)
        return h_new, c_new

    xh0_ref[:, :EMB] = emb_ref[0].astype(bf)
    w0 = w0_ref[...]
    w1 = w1_ref[...]
    b0 = b0_ref[0]
    b1 = b1_ref[0]

    h0s, c0s, h1s, c1s = [], [], [], []
    for n in range(NB):
        sl = pl.ds(n * BB, BB)
        h0, c0 = cell(xh0_ref[sl], c0_ref[sl], w0, b0)
        xh0_ref[sl, EMB:] = h0.astype(bf)
        c0_ref[sl] = c0
        xh1_ref[sl, :HID] = h0.astype(bf)
        h0s.append(h0)
        c0s.append(c0)
    for n in range(NB):
        sl = pl.ds(n * BB, BB)
        h1, c1 = cell(xh1_ref[sl], c1_ref[sl], w1, b1)
        xh1_ref[sl, HID:] = h1.astype(bf)
        c1_ref[sl] = c1
        h1s.append(h1)
        c1s.append(c1)

    @pl.when(t == SEQ - 1)
    def _emit():
        for n in range(NB):
            sl = pl.ds(n * BB, BB)
            hidden_ref[0, sl] = h0s[n]
            hidden_ref[1, sl] = h1s[n]
            cell_ref[0, sl] = c0s[n]
            cell_ref[1, sl] = c1s[n]


def _lstm_scan(emb_tm, w0, b0, w1, b1, interpret=False):
    full = lambda shape: pl.BlockSpec(shape, lambda t: (0,) * len(shape))
    return pl.pallas_call(
        _lstm_scan_kernel,
        grid=(SEQ,),
        in_specs=[
            pl.BlockSpec((1, BATCH, EMB), lambda t: (t, 0, 0)),
            full((K0, 4 * HID)),
            full((1, 4 * HID)),
            full((K1, 4 * HID)),
            full((1, 4 * HID)),
        ],
        out_specs=[
            full((2, BATCH, HID)),
            full((2, BATCH, HID)),
        ],
        out_shape=[
            jax.ShapeDtypeStruct((2, BATCH, HID), jnp.float32),
            jax.ShapeDtypeStruct((2, BATCH, HID), jnp.float32),
        ],
        scratch_shapes=[
            pltpu.VMEM((BATCH, K0), jnp.bfloat16),
            pltpu.VMEM((BATCH, HID), jnp.float32),
            pltpu.VMEM((BATCH, K1), jnp.bfloat16),
            pltpu.VMEM((BATCH, HID), jnp.float32),
        ],
        interpret=interpret,
    )(emb_tm, w0, b0, w1, b1)


def _pack_w(w_ih, w_hh, b_ih, b_hh):
    scale = jnp.concatenate([
        jnp.full((1, HID), 0.5), jnp.full((1, HID), 0.5),
        jnp.ones((1, HID)), jnp.full((1, HID), 0.5)], axis=1)
    w = jnp.concatenate([w_ih.T, w_hh.T], axis=0) * scale
    b = (b_ih + b_hh).reshape(1, -1) * scale
    return w.astype(jnp.bfloat16), b


@jax.jit
def kernel(table, W_ih_0, W_hh_0, b_ih_0, b_hh_0,
           W_ih_1, W_hh_1, b_ih_1, b_hh_1, x):
    idx = x.astype(jnp.int32).T.reshape(NW * N_CHUNK, CHUNK)
    emb = _make_sc_gather()(table, idx)
    emb_tm = emb.reshape(SEQ, BATCH, EMB)

    w0, b0 = _pack_w(W_ih_0, W_hh_0, b_ih_0, b_hh_0)
    w1, b1 = _pack_w(W_ih_1, W_hh_1, b_ih_1, b_hh_1)
    hidden, cell = _lstm_scan(emb_tm, w0, b0, w1, b1)
    return (hidden, cell)
